# baseline (speedup 1.0000x reference)
import functools

import jax
import jax.numpy as jnp
from jax import lax
from jax.experimental import pallas as pl
from jax.experimental.pallas import tpu as pltpu
from jax.experimental.pallas import tpu_sc as plsc

NC = 2
NS = 16
LANES = 16

F32 = jnp.float32
I32 = jnp.int32

_BCAST_DN = jax.lax.GatherDimensionNumbers(
    offset_dims=(), collapsed_slice_dims=(0,), start_index_map=(0,))
_PIB = jax.lax.GatherScatterMode.PROMISE_IN_BOUNDS


def _pq_body(x_ref, w_ref, out_ref):
    out_ref[...] = jnp.dot(x_ref[...], w_ref[...],
                           preferred_element_type=F32)


def _compute_pq(x, w12):
    n, f = x.shape
    bn = 2000
    assert n % bn == 0
    return pl.pallas_call(
        _pq_body,
        grid=(n // bn,),
        in_specs=[
            pl.BlockSpec((bn, f), lambda i: (i, 0)),
            pl.BlockSpec((f, 8), lambda i: (0, 0)),
        ],
        out_specs=pl.BlockSpec((bn, 8), lambda i: (i, 0)),
        out_shape=jax.ShapeDtypeStruct((n, 8), F32),
    )(x, w12)


def _alpha_kernel(n, e_total, cb):
    mesh = plsc.VectorSubcoreMesh(
        core_axis_name="c", subcore_axis_name="s",
        num_cores=NC, num_subcores=NS)
    nw = NC * NS
    per_w = e_total // nw
    assert per_w % cb == 0
    n_chunks = per_w // cb
    ng = cb // LANES

    @functools.partial(
        pl.kernel, mesh=mesh,
        out_type=jax.ShapeDtypeStruct((4 * e_total,), F32),
        compiler_params=pltpu.CompilerParams(needs_layout_passes=False),
        scratch_types=[
            pltpu.VMEM((n * 8,), F32),
            pltpu.VMEM((cb,), I32),
            pltpu.VMEM((cb,), I32),
            pltpu.VMEM((4, cb), F32),
            pltpu.VMEM((16,), F32),
        ],
    )
    def body(pq_hbm, col_hbm, row_hbm, b2p_hbm, alpha_hbm,
             pqv, colv, rowv, aout, b2v):
        wid = lax.axis_index("s") * NC + lax.axis_index("c")
        pltpu.sync_copy(b2p_hbm, b2v)
        pltpu.sync_copy(pq_hbm, pqv)
        b2c = [plsc.load_gather(b2v, [jnp.full((LANES,), c, I32)])
               for c in range(4)]
        iota = lax.iota(I32, LANES)

        def chunk(t, _):
            base = wid * per_w + t * cb
            pltpu.sync_copy(col_hbm.at[pl.ds(base, cb)], colv)
            pltpu.sync_copy(row_hbm.at[pl.ds(base, cb)], rowv)
            for g in range(ng):
                sl = pl.ds(g * LANES, LANES)
                cvec = colv[sl] * 8
                rvec = rowv[sl] * 8
                gs = []
                for c in range(4):
                    pc = plsc.load_gather(pqv, [cvec + c])
                    qc = plsc.load_gather(pqv, [rvec + (c + 4)])
                    gs.append(pc + qc + b2c[c])
                m = jnp.maximum(jnp.maximum(gs[0], gs[1]),
                                jnp.maximum(gs[2], gs[3]))
                es = [jnp.exp(gv - m) for gv in gs]
                inv = 1.0 / (es[0] + es[1] + es[2] + es[3])
                for c in range(4):
                    aout[c, sl] = es[c] * inv
            for c in range(4):
                pltpu.sync_copy(aout.at[c],
                                alpha_hbm.at[pl.ds(c * e_total + base, cb)])
            return ()

        lax.fori_loop(0, n_chunks, chunk, (), unroll=False)

    return body


def _scatter_kernel(n, feat, e_total, cb):
    mesh = plsc.VectorSubcoreMesh(
        core_axis_name="c", subcore_axis_name="s",
        num_cores=NC, num_subcores=NS)
    per_tile = e_total // NS
    assert per_tile % cb == 0
    n_chunks = per_tile // cb
    zrows = 40
    nz = n // zrows
    zfull, zrem = nz // NS, nz % NS
    drows = 200
    nd = n // drows
    dfull, drem = nd // NS, nd % NS
    nf = feat // LANES

    grp = 10
    assert n_chunks % grp == 0 and grp % 2 == 0
    n_groups = n_chunks // grp
    gsz = grp * cb

    @functools.partial(
        pl.kernel, mesh=mesh,
        out_type=jax.ShapeDtypeStruct((4, n, feat), F32),
        compiler_params=pltpu.CompilerParams(needs_layout_passes=False),
        scratch_types=[
            pltpu.VMEM_SHARED((n, feat), F32),
            pltpu.VMEM((2 * gsz,), I32),
            pltpu.VMEM((2 * gsz,), I32),
            pltpu.VMEM((2 * gsz,), F32),
            pltpu.VMEM((cb, feat), F32),
            pltpu.VMEM((cb, feat), F32),
            pltpu.VMEM((cb, feat), F32),
            pltpu.VMEM((cb, feat), F32),
            pltpu.VMEM((zrows, feat), F32),
            pltpu.SemaphoreType.DMA,
            pltpu.SemaphoreType.DMA,
            pltpu.SemaphoreType.DMA,
            pltpu.SemaphoreType.DMA,
        ],
    )
    def body(x_hbm, col_hbm, row_hbm, alpha_hbm, s_hbm,
             acc, colb, rowb, ab, rg0, rg1, rs0, rs1, zbuf,
             gs0, gs1, ss0, ss1):
        core = lax.axis_index("c")
        sid = lax.axis_index("s")
        zvec = jnp.zeros((LANES,), F32)
        rg = [rg0, rg1]
        rs = [rs0, rs1]
        gsem = [gs0, gs1]
        ssem = [ss0, ss1]
        tile_base = sid * per_tile

        def zrow(i, _):
            for j in range(nf):
                zbuf[i, pl.ds(j * LANES, LANES)] = zvec
            return ()
        lax.fori_loop(0, zrows, zrow, (), unroll=False)

        def start_gather(off, p):
            pltpu.async_copy(
                x_hbm.at[colb.at[pl.ds(off, cb)]], rg[p], gsem[p])

        def wait_gather(p):
            pltpu.make_async_copy(
                x_hbm.at[colb.at[pl.ds(0, cb)]], rg[p], gsem[p]).wait()

        def wait_scatter(p):
            pltpu.make_async_copy(
                rs[p], acc.at[rowb.at[pl.ds(0, cb)]], ssem[p]).wait()

        for kk in range(2):
            k = core * 2 + kk
            for r in range(zfull):
                off = (sid + r * NS) * zrows
                pltpu.sync_copy(zbuf, acc.at[pl.ds(off, zrows)])
            if zrem:
                @pl.when(sid < zrem)
                def _():
                    off = (zfull * NS + sid) * zrows
                    pltpu.sync_copy(zbuf, acc.at[pl.ds(off, zrows)])
            plsc.subcore_barrier()

            def load_idx(half_off, ebase):
                pltpu.sync_copy(col_hbm.at[pl.ds(ebase, gsz)],
                                colb.at[pl.ds(half_off, gsz)])
                pltpu.sync_copy(row_hbm.at[pl.ds(ebase, gsz)],
                                rowb.at[pl.ds(half_off, gsz)])
                pltpu.sync_copy(alpha_hbm.at[pl.ds(k * e_total + ebase,
                                                   gsz)],
                                ab.at[pl.ds(half_off, gsz)])

            load_idx(0, tile_base)
            start_gather(0, 0)
            start_gather(cb, 1)

            n_duos = grp // 2

            def group(g, _):
                sel = (g % 2) * gsz
                nxt = ((g + 1) % 2) * gsz

                def duo(d, _):
                    for u in range(2):
                        p = u
                        base = sel + (d * 2 + u) * cb
                        wait_gather(p)

                        @pl.when((g > 0) | (d > 0))
                        def _():
                            wait_scatter(p)

                        def scale_blk(gb, _):
                            avec = ab[pl.ds(base + gb * LANES, LANES)]

                            def srow(il, _):
                                asp = lax.gather(
                                    avec, jnp.full((LANES, 1), il, I32),
                                    _BCAST_DN, (1,), mode=_PIB)
                                i = gb * LANES + il
                                for f in range(nf):
                                    sl = pl.ds(f * LANES, LANES)
                                    rs[p][i, sl] = rg[p][i, sl] * asp
                                return ()
                            lax.fori_loop(0, LANES, srow, (), unroll=4)
                            return ()
                        lax.fori_loop(0, cb // LANES, scale_blk, (),
                                      unroll=False)

                        pltpu.async_copy(
                            rs[p], acc.at[rowb.at[pl.ds(base, cb)]],
                            ssem[p], add=True)

                        if u == 0:
                            @pl.when((d == 1) & (g < n_groups - 1))
                            def _():
                                load_idx(nxt, tile_base + (g + 1) * gsz)

                        @pl.when(d < n_duos - 1)
                        def _():
                            start_gather(base + 2 * cb, p)

                        @pl.when((d == n_duos - 1) & (g < n_groups - 1))
                        def _():
                            start_gather(nxt + u * cb, p)
                    return ()

                lax.fori_loop(0, n_duos, duo, (), unroll=False)
                return ()

            lax.fori_loop(0, n_groups, group, (), unroll=False)
            for p in range(2):
                wait_scatter(p)
            plsc.subcore_barrier()
            for r in range(dfull):
                off = (sid + r * NS) * drows
                pltpu.sync_copy(acc.at[pl.ds(off, drows)],
                                s_hbm.at[k, pl.ds(off, drows)])
            if drem:
                @pl.when(sid < drem)
                def _():
                    off = (dfull * NS + sid) * drows
                    pltpu.sync_copy(acc.at[pl.ds(off, drows)],
                                    s_hbm.at[k, pl.ds(off, drows)])
            plsc.subcore_barrier()

    return body


def _head_body(s_ref, wc_ref, b_ref, y1_ref, y2_ref):
    t = jnp.dot(s_ref[0], wc_ref[0], preferred_element_type=F32)
    t = t + b_ref[0, 0]
    nrm = jnp.sqrt(jnp.sum(t * t, axis=1, keepdims=True))
    y = t / jnp.maximum(nrm, 1e-12)
    y1_ref[...] = y
    y2_ref[0] = y


def _head(s, wc, bias):
    ch, n, f = s.shape
    h = wc.shape[2]
    bn = 1000
    assert n % bn == 0
    return pl.pallas_call(
        _head_body,
        grid=(ch, n // bn),
        in_specs=[
            pl.BlockSpec((1, bn, f), lambda k, i: (k, i, 0)),
            pl.BlockSpec((1, f, h), lambda k, i: (k, 0, 0)),
            pl.BlockSpec((1, 1, h), lambda k, i: (k, 0, 0)),
        ],
        out_specs=[
            pl.BlockSpec((bn, h), lambda k, i: (i, k)),
            pl.BlockSpec((1, bn, h), lambda k, i: (k, i, 0)),
        ],
        out_shape=[
            jax.ShapeDtypeStruct((n, ch * h), F32),
            jax.ShapeDtypeStruct((ch, n, h), F32),
        ],
    )(s, wc, bias)


def kernel(x, edge_index, W1, b1, W2, b2, Wc, bias):
    n, feat = x.shape
    e_total = edge_index.shape[1]
    ch = Wc.shape[0]
    assert ch == 4

    row = edge_index[0]
    col = edge_index[1]

    b12 = W1 @ W2
    w12 = jnp.concatenate([b12[:feat], b12[feat:]], axis=1)
    b2p = jnp.zeros((16,), F32).at[:4].set(b1 @ W2 + b2)

    pq = _compute_pq(x, w12).reshape(n * 8)
    alpha = _alpha_kernel(n, e_total, 80)(pq, col, row, b2p)
    s = _scatter_kernel(n, feat, e_total, 80)(x, col, row, alpha)
    y1, y2 = _head(s, Wc, bias.reshape(ch, 1, -1))

    output = y1
    outputs = y2[: ch // 2].reshape((ch // 2) * n, y2.shape[2])
    outputus = y2[ch // 2:].reshape((ch // 2) * n, y2.shape[2])
    return (output, outputs, outputus)


# speedup vs baseline: 2.3114x; 2.3114x over previous
import functools

import jax
import jax.numpy as jnp
from jax import lax
from jax.experimental import pallas as pl
from jax.experimental.pallas import tpu as pltpu
from jax.experimental.pallas import tpu_sc as plsc

NC = 2
NS = 16
LANES = 16

F32 = jnp.float32
I32 = jnp.int32

_BCAST_DN = jax.lax.GatherDimensionNumbers(
    offset_dims=(), collapsed_slice_dims=(0,), start_index_map=(0,))
_PIB = jax.lax.GatherScatterMode.PROMISE_IN_BOUNDS


def _pq_body(x_ref, w_ref, out_ref):
    out_ref[...] = jnp.dot(x_ref[...], w_ref[...],
                           preferred_element_type=F32)


def _compute_pq(x, w12):
    n, f = x.shape
    bn = 2000
    assert n % bn == 0
    return pl.pallas_call(
        _pq_body,
        grid=(n // bn,),
        in_specs=[
            pl.BlockSpec((bn, f), lambda i: (i, 0)),
            pl.BlockSpec((f, 8), lambda i: (0, 0)),
        ],
        out_specs=pl.BlockSpec((bn, 8), lambda i: (i, 0)),
        out_shape=jax.ShapeDtypeStruct((n, 8), F32),
    )(x, w12)


def _alpha_kernel(n, e_total, cb):
    mesh = plsc.VectorSubcoreMesh(
        core_axis_name="c", subcore_axis_name="s",
        num_cores=NC, num_subcores=NS)
    nw = NC * NS
    per_w = e_total // nw
    assert per_w % cb == 0
    n_chunks = per_w // cb
    ng = cb // LANES

    @functools.partial(
        pl.kernel, mesh=mesh,
        out_type=jax.ShapeDtypeStruct((4 * e_total,), F32),
        compiler_params=pltpu.CompilerParams(needs_layout_passes=False),
        scratch_types=[
            pltpu.VMEM((n * 8,), F32),
            pltpu.VMEM((cb,), I32),
            pltpu.VMEM((cb,), I32),
            pltpu.VMEM((4, cb), F32),
            pltpu.VMEM((16,), F32),
        ],
    )
    def body(pq_hbm, col_hbm, row_hbm, b2p_hbm, alpha_hbm,
             pqv, colv, rowv, aout, b2v):
        wid = lax.axis_index("s") * NC + lax.axis_index("c")
        pltpu.sync_copy(b2p_hbm, b2v)
        pltpu.sync_copy(pq_hbm, pqv)
        b2c = [plsc.load_gather(b2v, [jnp.full((LANES,), c, I32)])
               for c in range(4)]
        iota = lax.iota(I32, LANES)

        def chunk(t, _):
            base = wid * per_w + t * cb
            pltpu.sync_copy(col_hbm.at[pl.ds(base, cb)], colv)
            pltpu.sync_copy(row_hbm.at[pl.ds(base, cb)], rowv)
            for g in range(ng):
                sl = pl.ds(g * LANES, LANES)
                cvec = colv[sl] * 8
                rvec = rowv[sl] * 8
                gs = []
                for c in range(4):
                    pc = plsc.load_gather(pqv, [cvec + c])
                    qc = plsc.load_gather(pqv, [rvec + (c + 4)])
                    gs.append(pc + qc + b2c[c])
                m = jnp.maximum(jnp.maximum(gs[0], gs[1]),
                                jnp.maximum(gs[2], gs[3]))
                es = [jnp.exp(gv - m) for gv in gs]
                inv = 1.0 / (es[0] + es[1] + es[2] + es[3])
                for c in range(4):
                    aout[c, sl] = es[c] * inv
            for c in range(4):
                pltpu.sync_copy(aout.at[c],
                                alpha_hbm.at[pl.ds(c * e_total + base, cb)])
            return ()

        lax.fori_loop(0, n_chunks, chunk, (), unroll=False)

    return body


def _scatter_kernel(n, feat, e_total, cb):
    mesh = plsc.VectorSubcoreMesh(
        core_axis_name="c", subcore_axis_name="s",
        num_cores=NC, num_subcores=NS)
    per_tile = e_total // NS
    assert per_tile % cb == 0
    n_chunks = per_tile // cb
    zrows = 40
    nz = n // zrows
    zfull, zrem = nz // NS, nz % NS
    drows = 200
    nd = n // drows
    dfull, drem = nd // NS, nd % NS
    nf = feat // LANES

    grp = 10
    assert n_chunks % grp == 0 and grp % 2 == 0
    n_groups = n_chunks // grp
    gsz = grp * cb

    @functools.partial(
        pl.kernel, mesh=mesh,
        out_type=jax.ShapeDtypeStruct((4, n, feat), F32),
        compiler_params=pltpu.CompilerParams(needs_layout_passes=False),
        scratch_types=[
            pltpu.VMEM_SHARED((n, feat), F32),
            pltpu.VMEM((2 * gsz,), I32),
            pltpu.VMEM((2 * gsz,), I32),
            pltpu.VMEM((2 * gsz,), F32),
            pltpu.VMEM((cb, feat), F32),
            pltpu.VMEM((cb, feat), F32),
            pltpu.VMEM((cb, feat), F32),
            pltpu.VMEM((cb, feat), F32),
            pltpu.VMEM((zrows, feat), F32),
            pltpu.SemaphoreType.DMA,
            pltpu.SemaphoreType.DMA,
            pltpu.SemaphoreType.DMA,
            pltpu.SemaphoreType.DMA,
        ],
    )
    def body(x_hbm, col_hbm, row_hbm, alpha_hbm, s_hbm,
             acc, colb, rowb, ab, rg0, rg1, rs0, rs1, zbuf,
             gs0, gs1, ss0, ss1):
        core = lax.axis_index("c")
        sid = lax.axis_index("s")
        zvec = jnp.zeros((LANES,), F32)
        rg = [rg0, rg1]
        rs = [rs0, rs1]
        gsem = [gs0, gs1]
        ssem = [ss0, ss1]
        tile_base = sid * per_tile

        def zrow(i, _):
            for j in range(nf):
                zbuf[i, pl.ds(j * LANES, LANES)] = zvec
            return ()
        lax.fori_loop(0, zrows, zrow, (), unroll=False)

        def start_gather(off, p):
            pltpu.async_copy(
                x_hbm.at[colb.at[pl.ds(off, cb)]], rg[p], gsem[p])

        def wait_gather(p):
            pltpu.make_async_copy(
                x_hbm.at[colb.at[pl.ds(0, cb)]], rg[p], gsem[p]).wait()

        def wait_scatter(p):
            pltpu.make_async_copy(
                rs[p], acc.at[rowb.at[pl.ds(0, cb)]], ssem[p]).wait()

        for kk in range(2):
            k = core * 2 + kk
            for r in range(zfull):
                off = (sid + r * NS) * zrows
                pltpu.sync_copy(zbuf, acc.at[pl.ds(off, zrows)])
            if zrem:
                @pl.when(sid < zrem)
                def _():
                    off = (zfull * NS + sid) * zrows
                    pltpu.sync_copy(zbuf, acc.at[pl.ds(off, zrows)])
            plsc.subcore_barrier()

            def load_idx(half_off, ebase):
                pltpu.sync_copy(col_hbm.at[pl.ds(ebase, gsz)],
                                colb.at[pl.ds(half_off, gsz)])
                pltpu.sync_copy(row_hbm.at[pl.ds(ebase, gsz)],
                                rowb.at[pl.ds(half_off, gsz)])
                pltpu.sync_copy(alpha_hbm.at[pl.ds(k * e_total + ebase,
                                                   gsz)],
                                ab.at[pl.ds(half_off, gsz)])

            load_idx(0, tile_base)
            start_gather(0, 0)
            start_gather(cb, 1)

            n_duos = grp // 2

            def group(g, _):
                sel = (g % 2) * gsz
                nxt = ((g + 1) % 2) * gsz

                def duo(d, _):
                    for u in range(2):
                        p = u
                        base = sel + (d * 2 + u) * cb
                        wait_gather(p)

                        @pl.when((g > 0) | (d > 0))
                        def _():
                            wait_scatter(p)

                        @plsc.parallel_loop(0, cb, step=1, unroll=8)
                        def _scale(i):
                            asp = plsc.load_gather(
                                ab, [jnp.full((LANES,), base, I32) + i])
                            for f in range(nf):
                                sl = pl.ds(f * LANES, LANES)
                                rs[p][i, sl] = rg[p][i, sl] * asp

                        pltpu.async_copy(
                            rs[p], acc.at[rowb.at[pl.ds(base, cb)]],
                            ssem[p], add=True)

                        if u == 0:
                            @pl.when((d == 1) & (g < n_groups - 1))
                            def _():
                                load_idx(nxt, tile_base + (g + 1) * gsz)

                        @pl.when(d < n_duos - 1)
                        def _():
                            start_gather(base + 2 * cb, p)

                        @pl.when((d == n_duos - 1) & (g < n_groups - 1))
                        def _():
                            start_gather(nxt + u * cb, p)
                    return ()

                lax.fori_loop(0, n_duos, duo, (), unroll=False)
                return ()

            lax.fori_loop(0, n_groups, group, (), unroll=False)
            for p in range(2):
                wait_scatter(p)
            plsc.subcore_barrier()
            for r in range(dfull):
                off = (sid + r * NS) * drows
                pltpu.sync_copy(acc.at[pl.ds(off, drows)],
                                s_hbm.at[k, pl.ds(off, drows)])
            if drem:
                @pl.when(sid < drem)
                def _():
                    off = (dfull * NS + sid) * drows
                    pltpu.sync_copy(acc.at[pl.ds(off, drows)],
                                    s_hbm.at[k, pl.ds(off, drows)])
            plsc.subcore_barrier()

    return body


def _head_body(s_ref, wc_ref, b_ref, y1_ref, y2_ref):
    t = jnp.dot(s_ref[0], wc_ref[0], preferred_element_type=F32)
    t = t + b_ref[0, 0]
    nrm = jnp.sqrt(jnp.sum(t * t, axis=1, keepdims=True))
    y = t / jnp.maximum(nrm, 1e-12)
    y1_ref[...] = y
    y2_ref[0] = y


def _head(s, wc, bias):
    ch, n, f = s.shape
    h = wc.shape[2]
    bn = 1000
    assert n % bn == 0
    return pl.pallas_call(
        _head_body,
        grid=(ch, n // bn),
        in_specs=[
            pl.BlockSpec((1, bn, f), lambda k, i: (k, i, 0)),
            pl.BlockSpec((1, f, h), lambda k, i: (k, 0, 0)),
            pl.BlockSpec((1, 1, h), lambda k, i: (k, 0, 0)),
        ],
        out_specs=[
            pl.BlockSpec((bn, h), lambda k, i: (i, k)),
            pl.BlockSpec((1, bn, h), lambda k, i: (k, i, 0)),
        ],
        out_shape=[
            jax.ShapeDtypeStruct((n, ch * h), F32),
            jax.ShapeDtypeStruct((ch, n, h), F32),
        ],
    )(s, wc, bias)


def kernel(x, edge_index, W1, b1, W2, b2, Wc, bias):
    n, feat = x.shape
    e_total = edge_index.shape[1]
    ch = Wc.shape[0]
    assert ch == 4

    row = edge_index[0]
    col = edge_index[1]

    b12 = W1 @ W2
    w12 = jnp.concatenate([b12[:feat], b12[feat:]], axis=1)
    b2p = jnp.zeros((16,), F32).at[:4].set(b1 @ W2 + b2)

    pq = _compute_pq(x, w12).reshape(n * 8)
    alpha = _alpha_kernel(n, e_total, 80)(pq, col, row, b2p)
    s = _scatter_kernel(n, feat, e_total, 80)(x, col, row, alpha)
    y1, y2 = _head(s, Wc, bias.reshape(ch, 1, -1))

    output = y1
    outputs = y2[: ch // 2].reshape((ch // 2) * n, y2.shape[2])
    outputus = y2[ch // 2:].reshape((ch // 2) * n, y2.shape[2])
    return (output, outputs, outputus)


# speedup vs baseline: 2.8063x; 1.2141x over previous
import functools

import jax
import jax.numpy as jnp
from jax import lax
from jax.experimental import pallas as pl
from jax.experimental.pallas import tpu as pltpu
from jax.experimental.pallas import tpu_sc as plsc

NC = 2
NS = 16
LANES = 16

F32 = jnp.float32
I32 = jnp.int32

_BCAST_DN = jax.lax.GatherDimensionNumbers(
    offset_dims=(), collapsed_slice_dims=(0,), start_index_map=(0,))
_PIB = jax.lax.GatherScatterMode.PROMISE_IN_BOUNDS


def _pq_body(x_ref, w_ref, out_ref):
    out_ref[...] = jnp.dot(x_ref[...], w_ref[...],
                           preferred_element_type=F32)


def _compute_pq(x, w12):
    n, f = x.shape
    bn = 2000
    assert n % bn == 0
    return pl.pallas_call(
        _pq_body,
        grid=(n // bn,),
        in_specs=[
            pl.BlockSpec((bn, f), lambda i: (i, 0)),
            pl.BlockSpec((f, 8), lambda i: (0, 0)),
        ],
        out_specs=pl.BlockSpec((bn, 8), lambda i: (i, 0)),
        out_shape=jax.ShapeDtypeStruct((n, 8), F32),
    )(x, w12)


def _alpha_kernel(n, e_total, bsz):
    mesh = plsc.VectorSubcoreMesh(
        core_axis_name="c", subcore_axis_name="s",
        num_cores=NC, num_subcores=NS)
    nw = NC * NS
    per_w = e_total // nw
    assert per_w % bsz == 0
    n_batches = per_w // bsz

    @functools.partial(
        pl.kernel, mesh=mesh,
        out_type=jax.ShapeDtypeStruct((4 * e_total,), F32),
        compiler_params=pltpu.CompilerParams(needs_layout_passes=False),
        scratch_types=[
            pltpu.VMEM((n * 8,), F32),
            pltpu.VMEM((bsz,), I32),
            pltpu.VMEM((bsz,), I32),
            pltpu.VMEM((bsz,), F32),
            pltpu.VMEM((bsz,), F32),
            pltpu.VMEM((bsz,), F32),
            pltpu.VMEM((bsz,), F32),
            pltpu.VMEM((16,), F32),
        ],
    )
    def body(pq_hbm, col_hbm, row_hbm, b2p_hbm, alpha_hbm,
             pqv, colv, rowv, a0, a1, a2, a3, b2v):
        aout = [a0, a1, a2, a3]
        wid = lax.axis_index("s") * NC + lax.axis_index("c")
        pltpu.sync_copy(b2p_hbm, b2v)
        pltpu.sync_copy(pq_hbm, pqv)
        b2c = [plsc.load_gather(b2v, [jnp.full((LANES,), c, I32)])
               for c in range(4)]

        def batch(t, _):
            base = wid * per_w + t * bsz
            pltpu.sync_copy(col_hbm.at[pl.ds(base, bsz)], colv)
            pltpu.sync_copy(row_hbm.at[pl.ds(base, bsz)], rowv)

            @plsc.parallel_loop(0, bsz // LANES, step=1, unroll=4)
            def _group(g):
                sl = pl.ds(g * LANES, LANES)
                cvec = colv[sl] * 8
                rvec = rowv[sl] * 8
                gs = []
                for c in range(4):
                    pc = plsc.load_gather(pqv, [cvec + c])
                    qc = plsc.load_gather(pqv, [rvec + (c + 4)])
                    gs.append(pc + qc + b2c[c])
                m = jnp.maximum(jnp.maximum(gs[0], gs[1]),
                                jnp.maximum(gs[2], gs[3]))
                es = [jnp.exp(gv - m) for gv in gs]
                inv = 1.0 / (es[0] + es[1] + es[2] + es[3])
                for c in range(4):
                    aout[c][sl] = es[c] * inv

            for c in range(4):
                pltpu.sync_copy(aout[c],
                                alpha_hbm.at[pl.ds(c * e_total + base,
                                                   bsz)])
            return ()

        lax.fori_loop(0, n_batches, batch, (), unroll=False)

    return body


def _scatter_kernel(n, feat, e_total, cb):
    mesh = plsc.VectorSubcoreMesh(
        core_axis_name="c", subcore_axis_name="s",
        num_cores=NC, num_subcores=NS)
    per_tile = e_total // NS
    assert per_tile % cb == 0
    n_chunks = per_tile // cb
    zrows = 40
    nz = n // zrows
    zfull, zrem = nz // NS, nz % NS
    drows = 200
    nd = n // drows
    dfull, drem = nd // NS, nd % NS
    nf = feat // LANES

    grp = 10
    assert n_chunks % grp == 0 and grp % 2 == 0
    n_groups = n_chunks // grp
    gsz = grp * cb

    @functools.partial(
        pl.kernel, mesh=mesh,
        out_type=jax.ShapeDtypeStruct((4, n, feat), F32),
        compiler_params=pltpu.CompilerParams(needs_layout_passes=False),
        scratch_types=[
            pltpu.VMEM_SHARED((n, feat), F32),
            pltpu.VMEM((2 * gsz,), I32),
            pltpu.VMEM((2 * gsz,), I32),
            pltpu.VMEM((2 * gsz,), F32),
            pltpu.VMEM((cb, feat), F32),
            pltpu.VMEM((cb, feat), F32),
            pltpu.VMEM((cb, feat), F32),
            pltpu.VMEM((cb, feat), F32),
            pltpu.VMEM((zrows, feat), F32),
            pltpu.SemaphoreType.DMA,
            pltpu.SemaphoreType.DMA,
            pltpu.SemaphoreType.DMA,
            pltpu.SemaphoreType.DMA,
        ],
    )
    def body(x_hbm, col_hbm, row_hbm, alpha_hbm, s_hbm,
             acc, colb, rowb, ab, rg0, rg1, rs0, rs1, zbuf,
             gs0, gs1, ss0, ss1):
        core = lax.axis_index("c")
        sid = lax.axis_index("s")
        zvec = jnp.zeros((LANES,), F32)
        rg = [rg0, rg1]
        rs = [rs0, rs1]
        gsem = [gs0, gs1]
        ssem = [ss0, ss1]
        tile_base = sid * per_tile

        def zrow(i, _):
            for j in range(nf):
                zbuf[i, pl.ds(j * LANES, LANES)] = zvec
            return ()
        lax.fori_loop(0, zrows, zrow, (), unroll=False)

        def start_gather(off, p):
            pltpu.async_copy(
                x_hbm.at[colb.at[pl.ds(off, cb)]], rg[p], gsem[p])

        def wait_gather(p):
            pltpu.make_async_copy(
                x_hbm.at[colb.at[pl.ds(0, cb)]], rg[p], gsem[p]).wait()

        def wait_scatter(p):
            pltpu.make_async_copy(
                rs[p], acc.at[rowb.at[pl.ds(0, cb)]], ssem[p]).wait()

        for kk in range(2):
            k = core * 2 + kk
            for r in range(zfull):
                off = (sid + r * NS) * zrows
                pltpu.sync_copy(zbuf, acc.at[pl.ds(off, zrows)])
            if zrem:
                @pl.when(sid < zrem)
                def _():
                    off = (zfull * NS + sid) * zrows
                    pltpu.sync_copy(zbuf, acc.at[pl.ds(off, zrows)])
            plsc.subcore_barrier()

            def load_idx(half_off, ebase):
                pltpu.sync_copy(col_hbm.at[pl.ds(ebase, gsz)],
                                colb.at[pl.ds(half_off, gsz)])
                pltpu.sync_copy(row_hbm.at[pl.ds(ebase, gsz)],
                                rowb.at[pl.ds(half_off, gsz)])
                pltpu.sync_copy(alpha_hbm.at[pl.ds(k * e_total + ebase,
                                                   gsz)],
                                ab.at[pl.ds(half_off, gsz)])

            load_idx(0, tile_base)
            start_gather(0, 0)
            start_gather(cb, 1)

            n_duos = grp // 2

            def group(g, _):
                sel = (g % 2) * gsz
                nxt = ((g + 1) % 2) * gsz

                def duo(d, _):
                    for u in range(2):
                        p = u
                        base = sel + (d * 2 + u) * cb
                        wait_gather(p)

                        @pl.when((g > 0) | (d > 0))
                        def _():
                            wait_scatter(p)

                        @plsc.parallel_loop(0, cb, step=1, unroll=8)
                        def _scale(i):
                            asp = plsc.load_gather(
                                ab, [jnp.full((LANES,), base, I32) + i])
                            for f in range(nf):
                                sl = pl.ds(f * LANES, LANES)
                                rs[p][i, sl] = rg[p][i, sl] * asp

                        pltpu.async_copy(
                            rs[p], acc.at[rowb.at[pl.ds(base, cb)]],
                            ssem[p], add=True)

                        if u == 0:
                            @pl.when((d == 1) & (g < n_groups - 1))
                            def _():
                                load_idx(nxt, tile_base + (g + 1) * gsz)

                        @pl.when(d < n_duos - 1)
                        def _():
                            start_gather(base + 2 * cb, p)

                        @pl.when((d == n_duos - 1) & (g < n_groups - 1))
                        def _():
                            start_gather(nxt + u * cb, p)
                    return ()

                lax.fori_loop(0, n_duos, duo, (), unroll=False)
                return ()

            lax.fori_loop(0, n_groups, group, (), unroll=False)
            for p in range(2):
                wait_scatter(p)
            plsc.subcore_barrier()
            for r in range(dfull):
                off = (sid + r * NS) * drows
                pltpu.sync_copy(acc.at[pl.ds(off, drows)],
                                s_hbm.at[k, pl.ds(off, drows)])
            if drem:
                @pl.when(sid < drem)
                def _():
                    off = (dfull * NS + sid) * drows
                    pltpu.sync_copy(acc.at[pl.ds(off, drows)],
                                    s_hbm.at[k, pl.ds(off, drows)])
            plsc.subcore_barrier()

    return body


def _head_body(s_ref, wc_ref, b_ref, y1_ref, y2_ref):
    t = jnp.dot(s_ref[0], wc_ref[0], preferred_element_type=F32)
    t = t + b_ref[0, 0]
    nrm = jnp.sqrt(jnp.sum(t * t, axis=1, keepdims=True))
    y = t / jnp.maximum(nrm, 1e-12)
    y1_ref[...] = y
    y2_ref[0] = y


def _head(s, wc, bias):
    ch, n, f = s.shape
    h = wc.shape[2]
    bn = 1000
    assert n % bn == 0
    return pl.pallas_call(
        _head_body,
        grid=(ch, n // bn),
        in_specs=[
            pl.BlockSpec((1, bn, f), lambda k, i: (k, i, 0)),
            pl.BlockSpec((1, f, h), lambda k, i: (k, 0, 0)),
            pl.BlockSpec((1, 1, h), lambda k, i: (k, 0, 0)),
        ],
        out_specs=[
            pl.BlockSpec((bn, h), lambda k, i: (i, k)),
            pl.BlockSpec((1, bn, h), lambda k, i: (k, i, 0)),
        ],
        out_shape=[
            jax.ShapeDtypeStruct((n, ch * h), F32),
            jax.ShapeDtypeStruct((ch, n, h), F32),
        ],
    )(s, wc, bias)


def kernel(x, edge_index, W1, b1, W2, b2, Wc, bias):
    n, feat = x.shape
    e_total = edge_index.shape[1]
    ch = Wc.shape[0]
    assert ch == 4

    row = edge_index[0]
    col = edge_index[1]

    b12 = W1 @ W2
    w12 = jnp.concatenate([b12[:feat], b12[feat:]], axis=1)
    b2p = jnp.zeros((16,), F32).at[:4].set(b1 @ W2 + b2)

    pq = _compute_pq(x, w12).reshape(n * 8)
    alpha = _alpha_kernel(n, e_total, 2000)(pq, col, row, b2p)
    s = _scatter_kernel(n, feat, e_total, 80)(x, col, row, alpha)
    y1, y2 = _head(s, Wc, bias.reshape(ch, 1, -1))

    output = y1
    outputs = y2[: ch // 2].reshape((ch // 2) * n, y2.shape[2])
    outputus = y2[ch // 2:].reshape((ch // 2) * n, y2.shape[2])
    return (output, outputs, outputus)
